# baseline (speedup 1.0000x reference)
import jax
import jax.numpy as jnp
from jax.experimental import pallas as pl
from jax.experimental.pallas import tpu as pltpu

_BM = 560
_H = _BM // 2


def _half(a, xf, xi, w, d_in):
    acc = jnp.dot(a.astype(jnp.bfloat16), xf, preferred_element_type=jnp.float32)
    deg = jnp.sum(a, axis=1, keepdims=True)
    agg = acc / jnp.clip(deg, 1e-6, None)
    return (jnp.dot(xi, w[:d_in], preferred_element_type=jnp.float32)
            + jnp.dot(agg, w[d_in:], preferred_element_type=jnp.float32))


def _body(xf_ref, adj0_ref, adj1_ref, xi_ref, w_ref, out_ref):
    d_in = xi_ref.shape[1]
    xf = xf_ref[...].astype(jnp.bfloat16)
    w = w_ref[...]
    out_ref[: _H, :] = _half(adj0_ref[...], xf, xi_ref[: _H, :], w, d_in)
    out_ref[_H:, :] = _half(adj1_ref[...], xf, xi_ref[_H:, :], w, d_in)


def kernel(x, adj, W):
    n, d_in = x.shape
    d_out = W.shape[1]
    nm = pl.cdiv(n, _BM)

    return pl.pallas_call(
        _body,
        grid=(nm,),
        in_specs=[
            pl.BlockSpec((n, d_in), lambda i: (0, 0)),
            pl.BlockSpec((_H, n), lambda i: (2 * i, 0)),
            pl.BlockSpec((_H, n), lambda i: (2 * i + 1, 0)),
            pl.BlockSpec((_BM, d_in), lambda i: (i, 0)),
            pl.BlockSpec((2 * d_in, d_out), lambda i: (0, 0)),
        ],
        out_specs=pl.BlockSpec((_BM, d_out), lambda i: (i, 0)),
        out_shape=jax.ShapeDtypeStruct((n, d_out), jnp.float32),
        compiler_params=pltpu.CompilerParams(
            dimension_semantics=("parallel",),
            vmem_limit_bytes=64 * 1024 * 1024,
        ),
    )(x, adj, adj, x, W)


# speedup vs baseline: 1.0543x; 1.0543x over previous
import jax
import jax.numpy as jnp
from jax.experimental import pallas as pl
from jax.experimental.pallas import tpu as pltpu

_BM = 400
_H = _BM // 2


def _half(a, xf, xi, w, d_in):
    acc = jnp.dot(a.astype(jnp.bfloat16), xf, preferred_element_type=jnp.float32)
    deg = jnp.sum(a, axis=1, keepdims=True)
    agg = acc / jnp.clip(deg, 1e-6, None)
    return (jnp.dot(xi, w[:d_in], preferred_element_type=jnp.float32)
            + jnp.dot(agg, w[d_in:], preferred_element_type=jnp.float32))


def _body(xf_ref, adj0_ref, adj1_ref, xi_ref, w_ref, out_ref):
    d_in = xi_ref.shape[1]
    xf = xf_ref[...].astype(jnp.bfloat16)
    w = w_ref[...]
    out_ref[: _H, :] = _half(adj0_ref[...], xf, xi_ref[: _H, :], w, d_in)
    out_ref[_H:, :] = _half(adj1_ref[...], xf, xi_ref[_H:, :], w, d_in)


def kernel(x, adj, W):
    n, d_in = x.shape
    d_out = W.shape[1]
    nm = pl.cdiv(n, _BM)

    return pl.pallas_call(
        _body,
        grid=(nm,),
        in_specs=[
            pl.BlockSpec((n, d_in), lambda i: (0, 0)),
            pl.BlockSpec((_H, n), lambda i: (2 * i, 0)),
            pl.BlockSpec((_H, n), lambda i: (2 * i + 1, 0)),
            pl.BlockSpec((_BM, d_in), lambda i: (i, 0)),
            pl.BlockSpec((2 * d_in, d_out), lambda i: (0, 0)),
        ],
        out_specs=pl.BlockSpec((_BM, d_out), lambda i: (i, 0)),
        out_shape=jax.ShapeDtypeStruct((n, d_out), jnp.float32),
        compiler_params=pltpu.CompilerParams(
            dimension_semantics=("parallel",),
            vmem_limit_bytes=64 * 1024 * 1024,
        ),
    )(x, adj, adj, x, W)


# speedup vs baseline: 1.0571x; 1.0026x over previous
import jax
import jax.numpy as jnp
from jax.experimental import pallas as pl
from jax.experimental.pallas import tpu as pltpu

_BM = 480
_H = _BM // 2


def _half(a, xf, xi, w, d_in):
    acc = jnp.dot(a.astype(jnp.bfloat16), xf, preferred_element_type=jnp.float32)
    deg = jnp.sum(a, axis=1, keepdims=True)
    agg = acc / jnp.clip(deg, 1e-6, None)
    return (jnp.dot(xi, w[:d_in], preferred_element_type=jnp.float32)
            + jnp.dot(agg, w[d_in:], preferred_element_type=jnp.float32))


def _body(xf_ref, adj0_ref, adj1_ref, xi_ref, w_ref, out_ref):
    d_in = xi_ref.shape[1]
    xf = xf_ref[...].astype(jnp.bfloat16)
    w = w_ref[...]
    out_ref[: _H, :] = _half(adj0_ref[...], xf, xi_ref[: _H, :], w, d_in)
    out_ref[_H:, :] = _half(adj1_ref[...], xf, xi_ref[_H:, :], w, d_in)


def kernel(x, adj, W):
    n, d_in = x.shape
    d_out = W.shape[1]
    nm = pl.cdiv(n, _BM)

    return pl.pallas_call(
        _body,
        grid=(nm,),
        in_specs=[
            pl.BlockSpec((n, d_in), lambda i: (0, 0)),
            pl.BlockSpec((_H, n), lambda i: (2 * i, 0)),
            pl.BlockSpec((_H, n), lambda i: (2 * i + 1, 0)),
            pl.BlockSpec((_BM, d_in), lambda i: (i, 0)),
            pl.BlockSpec((2 * d_in, d_out), lambda i: (0, 0)),
        ],
        out_specs=pl.BlockSpec((_BM, d_out), lambda i: (i, 0)),
        out_shape=jax.ShapeDtypeStruct((n, d_out), jnp.float32),
        compiler_params=pltpu.CompilerParams(
            dimension_semantics=("parallel",),
            vmem_limit_bytes=64 * 1024 * 1024,
        ),
    )(x, adj, adj, x, W)


# speedup vs baseline: 1.0700x; 1.0122x over previous
import jax
import jax.numpy as jnp
from jax.experimental import pallas as pl
from jax.experimental.pallas import tpu as pltpu

_BM = 400
_H = _BM // 2


def _half(a, xf, xi, w, d_in):
    acc = jnp.dot(a.astype(jnp.bfloat16), xf, preferred_element_type=jnp.float32)
    deg = jnp.sum(a, axis=1, keepdims=True)
    agg = acc / jnp.clip(deg, 1e-6, None)
    return (jnp.dot(xi, w[:d_in], preferred_element_type=jnp.float32)
            + jnp.dot(agg, w[d_in:], preferred_element_type=jnp.float32))


def _body(xf_ref, adj0_ref, adj1_ref, w_ref, out_ref):
    i = pl.program_id(0)
    d_in = xf_ref.shape[1]
    xf = xf_ref[...].astype(jnp.bfloat16)
    w = w_ref[...]
    xi0 = xf_ref[pl.ds(i * _BM, _H), :]
    xi1 = xf_ref[pl.ds(i * _BM + _H, _H), :]
    out_ref[: _H, :] = _half(adj0_ref[...], xf, xi0, w, d_in)
    out_ref[_H:, :] = _half(adj1_ref[...], xf, xi1, w, d_in)


def kernel(x, adj, W):
    n, d_in = x.shape
    d_out = W.shape[1]
    nm = pl.cdiv(n, _BM)

    return pl.pallas_call(
        _body,
        grid=(nm,),
        in_specs=[
            pl.BlockSpec((n, d_in), lambda i: (0, 0)),
            pl.BlockSpec((_H, n), lambda i: (2 * i, 0)),
            pl.BlockSpec((_H, n), lambda i: (2 * i + 1, 0)),
            pl.BlockSpec((2 * d_in, d_out), lambda i: (0, 0)),
        ],
        out_specs=pl.BlockSpec((_BM, d_out), lambda i: (i, 0)),
        out_shape=jax.ShapeDtypeStruct((n, d_out), jnp.float32),
        compiler_params=pltpu.CompilerParams(
            dimension_semantics=("parallel",),
            vmem_limit_bytes=64 * 1024 * 1024,
        ),
    )(x, adj, adj, W)
